# baseline (speedup 1.0000x reference)
import functools

import jax
import jax.numpy as jnp
from jax import lax
from jax.experimental import pallas as pl
from jax.experimental.pallas import tpu as pltpu
from jax.experimental.pallas import tpu_sc as plsc

N_TEST = 1024
N_TRAIN = 100000
FEAT = 64
KNN = 16
LANE = 128
TILE = 2048
N_PAD = 100352
N_TILES = N_PAD // TILE
CPT = TILE // LANE
N_CHUNKS = N_PAD // LANE
QBLK = 128
N_QBLK = N_TEST // QBLK
CAND = KNN * LANE
NEG = -1.0e30
NC = 2
NS = 16
NW = NC * NS
ROWS_W = (N_TEST * KNN) // NW
N_SUB = ROWS_W // LANE


def _k1_body(a_ref, a2_ref, b_ref, s_ref, ids_ref, m_scr, yn_scr):
    j = pl.program_id(0)
    i = pl.program_id(1)
    row0 = pl.multiple_of(i * QBLK, QBLK)
    a = a_ref[pl.ds(row0, QBLK), :]
    a2 = a2_ref[pl.ds(row0, QBLK), :]
    b = b_ref[...]

    @pl.when(i == 0)
    def _():
        yn_scr[...] = jnp.sum(b * b, axis=1)[None, :]

    dots2 = lax.dot_general(a2, b, (((1,), (1,)), ((), ())),
                            preferred_element_type=jnp.float32)
    xn = jnp.sum(a * a, axis=1, keepdims=True)
    yn = yn_scr[...]
    scores = dots2 - (xn + yn)
    for c in range(CPT):
        s_ref[:, c, :] = scores[:, c * LANE:(c + 1) * LANE]

    tmax = jnp.max(scores.reshape(QBLK, CPT, LANE), axis=2)
    m_scr[i, j] = jnp.transpose(tmax)

    @pl.when(j == N_TILES - 1)
    def _():
        m = m_scr[i].reshape(N_CHUNKS, QBLK)
        iota = lax.broadcasted_iota(jnp.int32, (N_CHUNKS, QBLK), 0)
        rows = []
        for _ in range(KNN):
            mx = jnp.max(m, axis=0, keepdims=True)
            pos = jnp.min(jnp.where(m == mx, iota, N_CHUNKS),
                          axis=0, keepdims=True)
            rows.append(pos)
            m = jnp.where(iota == pos, NEG, m)
        ids_ref[...] = jnp.concatenate(rows, axis=0)


def _k1(h_test, h_test2, h_train_pad):
    return pl.pallas_call(
        _k1_body,
        grid=(N_TILES, N_QBLK),
        in_specs=[
            pl.BlockSpec((N_TEST, FEAT), lambda j, i: (0, 0)),
            pl.BlockSpec((N_TEST, FEAT), lambda j, i: (0, 0)),
            pl.BlockSpec((TILE, FEAT), lambda j, i: (j, 0)),
        ],
        out_specs=[
            pl.BlockSpec((QBLK, CPT, LANE), lambda j, i: (i, j, 0)),
            pl.BlockSpec((KNN, QBLK), lambda j, i: (0, i)),
        ],
        out_shape=[
            jax.ShapeDtypeStruct((N_TEST, N_CHUNKS, LANE), jnp.float32),
            jax.ShapeDtypeStruct((KNN, N_TEST), jnp.int32),
        ],
        scratch_shapes=[
            pltpu.VMEM((N_QBLK, N_TILES, CPT, QBLK), jnp.float32),
            pltpu.VMEM((1, TILE), jnp.float32),
        ],
        compiler_params=pltpu.CompilerParams(
            dimension_semantics=("arbitrary", "arbitrary")),
    )(h_test, h_test2, h_train_pad)


def _sc_gather(t_s, t_p0, t_p1, idx_s, idx_p):
    n_rows = N_TEST * KNN
    mesh = plsc.VectorSubcoreMesh(core_axis_name="c", subcore_axis_name="s")

    nbuf = 4

    @functools.partial(
        pl.kernel, mesh=mesh,
        out_type=[
            jax.ShapeDtypeStruct((n_rows, LANE), jnp.float32),
            jax.ShapeDtypeStruct((n_rows, LANE), jnp.float32),
            jax.ShapeDtypeStruct((n_rows, LANE), jnp.float32),
        ],
        scratch_types=(
            [pltpu.VMEM((N_SUB, LANE), jnp.int32),
             pltpu.VMEM((N_SUB, LANE), jnp.int32)]
            + [pltpu.VMEM((LANE, LANE), jnp.float32)] * nbuf
            + [pltpu.SemaphoreType.DMA] * (2 * nbuf)
        ),
    )
    def k2(ts_hbm, tp0_hbm, tp1_hbm, ixs_hbm, ixp_hbm,
           o_s, o_p0, o_p1, ixs_v, ixp_v, *bufsem):
        bufs = bufsem[:nbuf]
        sin = bufsem[nbuf:2 * nbuf]
        sout = bufsem[2 * nbuf:]
        wid = lax.axis_index("s") * NC + lax.axis_index("c")
        pltpu.sync_copy(ixs_hbm.at[wid], ixs_v)
        pltpu.sync_copy(ixp_hbm.at[wid], ixp_v)
        tasks = []
        for t in range(N_SUB):
            tasks.append((ts_hbm, ixs_v, o_s, t))
            tasks.append((tp0_hbm, ixp_v, o_p0, t))
            tasks.append((tp1_hbm, ixp_v, o_p1, t))
        n_tasks = len(tasks)
        inh = [None] * n_tasks
        outh = [None] * n_tasks

        def start_out(k):
            table, ixv, out, t = tasks[k]
            b = k % nbuf
            inh[k].wait()
            base = wid * ROWS_W + t * LANE
            outh[k] = pltpu.async_copy(bufs[b], out.at[pl.ds(base, LANE)],
                                       sout[b])

        for k in range(n_tasks):
            b = k % nbuf
            if k >= nbuf:
                outh[k - nbuf].wait()
            table, ixv, out, t = tasks[k]
            inh[k] = pltpu.async_copy(table.at[ixv.at[t]], bufs[b], sin[b])
            if k >= 1:
                start_out(k - 1)
        start_out(n_tasks - 1)
        for k in range(n_tasks - nbuf, n_tasks):
            outh[k].wait()

    return k2(t_s, t_p0, t_p1, idx_s, idx_p)


def _k3_body(s_ref, p0_ref, p1_ref, o_ref):
    s = s_ref[...]
    iota = lax.broadcasted_iota(jnp.int32, (QBLK, CAND), 1)
    msk = jnp.zeros((QBLK, CAND), jnp.float32)
    for _ in range(KNN):
        mx = jnp.max(s, axis=1, keepdims=True)
        pos = jnp.min(jnp.where(s == mx, iota, CAND), axis=1, keepdims=True)
        hit = iota == pos
        msk = jnp.where(hit, 1.0, msk)
        s = jnp.where(hit, NEG, s)
    o0 = jnp.sum(p0_ref[...] * msk, axis=1) * (1.0 / KNN)
    o1 = jnp.sum(p1_ref[...] * msk, axis=1) * (1.0 / KNN)
    zero = jnp.zeros((6, QBLK), jnp.float32)
    o_ref[...] = jnp.concatenate([o0[None], o1[None], zero], axis=0)


def _k3(cand_s, cand_p0, cand_p1):
    return pl.pallas_call(
        _k3_body,
        grid=(N_QBLK,),
        in_specs=[
            pl.BlockSpec((QBLK, CAND), lambda i: (i, 0)),
            pl.BlockSpec((QBLK, CAND), lambda i: (i, 0)),
            pl.BlockSpec((QBLK, CAND), lambda i: (i, 0)),
        ],
        out_specs=pl.BlockSpec((8, QBLK), lambda i: (0, i)),
        out_shape=jax.ShapeDtypeStruct((8, N_TEST), jnp.float32),
        compiler_params=pltpu.CompilerParams(
            dimension_semantics=("parallel",)),
    )(cand_s, cand_p0, cand_p1)


def kernel(H_test, H_train, p_hat_train, K):
    del K
    pad_rows = jnp.zeros((N_PAD - N_TRAIN, FEAT), jnp.float32)
    pad_rows = pad_rows.at[:, 0].set(1.0e18)
    h_train_pad = jnp.concatenate([H_train, pad_rows], axis=0)
    scores, ids_t = _k1(H_test, H_test * 2.0, h_train_pad)
    chunk_ids = jnp.transpose(ids_t)

    idx_p = chunk_ids.reshape(NW, N_SUB, LANE)
    idx_s = (jnp.arange(N_TEST, dtype=jnp.int32)[:, None] * N_CHUNKS
             + chunk_ids).reshape(NW, N_SUB, LANE)

    t_s = scores.reshape(N_TEST * N_CHUNKS, LANE)
    pp = jnp.pad(p_hat_train, ((0, 0), (0, N_PAD - N_TRAIN)))
    t_p0 = pp[0].reshape(N_CHUNKS, LANE)
    t_p1 = pp[1].reshape(N_CHUNKS, LANE)

    cand_s, cand_p0, cand_p1 = _sc_gather(t_s, t_p0, t_p1, idx_s, idx_p)
    out = _k3(cand_s.reshape(N_TEST, CAND),
              cand_p0.reshape(N_TEST, CAND),
              cand_p1.reshape(N_TEST, CAND))
    return out[:2, :]


# speedup vs baseline: 1.1261x; 1.1261x over previous
import functools

import jax
import jax.numpy as jnp
from jax import lax
from jax.experimental import pallas as pl
from jax.experimental.pallas import tpu as pltpu
from jax.experimental.pallas import tpu_sc as plsc

N_TEST = 1024
N_TRAIN = 100000
FEAT = 64
KNN = 16
LANE = 128
TILE = 2048
N_PAD = 100352
N_TILES = N_PAD // TILE
CPT = TILE // LANE
N_CHUNKS = N_PAD // LANE
QBLK = 128
N_QBLK = N_TEST // QBLK
CAND = KNN * LANE
NEG = -1.0e30
NC = 2
NS = 16
NW = NC * NS
ROWS_W = (N_TEST * KNN) // NW
N_SUB = ROWS_W // LANE


def _k1_body(a_ref, a2_ref, b_ref, s_ref, ids_ref, m_scr, yn_scr):
    j = pl.program_id(0)
    i = pl.program_id(1)
    row0 = pl.multiple_of(i * QBLK, QBLK)
    a = a_ref[pl.ds(row0, QBLK), :]
    a2 = a2_ref[pl.ds(row0, QBLK), :]
    b = b_ref[...]

    @pl.when(i == 0)
    def _():
        yn_scr[...] = jnp.sum(b * b, axis=1)[None, :]

    xn = jnp.sum(a * a, axis=1, keepdims=True)
    tcols = []
    for c in range(CPT):
        bc = b[c * LANE:(c + 1) * LANE, :]
        yn_c = yn_scr[:, c * LANE:(c + 1) * LANE]
        dots2 = lax.dot_general(a2, bc, (((1,), (1,)), ((), ())),
                                preferred_element_type=jnp.float32)
        sc = dots2 - (xn + yn_c)
        s_ref[:, c, :] = sc
        tcols.append(jnp.max(sc, axis=1, keepdims=True))
    tmax = jnp.concatenate(tcols, axis=1)
    m_scr[i, j] = jnp.transpose(tmax)

    @pl.when(j == N_TILES - 1)
    def _():
        m = m_scr[i].reshape(N_CHUNKS, QBLK)
        iota = lax.broadcasted_iota(jnp.int32, (N_CHUNKS, QBLK), 0)
        rows = []
        for _ in range(KNN):
            mx = jnp.max(m, axis=0, keepdims=True)
            pos = jnp.min(jnp.where(m == mx, iota, N_CHUNKS),
                          axis=0, keepdims=True)
            rows.append(pos)
            m = jnp.where(iota == pos, NEG, m)
        ids_ref[...] = jnp.concatenate(rows, axis=0)


def _k1(h_test, h_test2, h_train_pad):
    return pl.pallas_call(
        _k1_body,
        grid=(N_TILES, N_QBLK),
        in_specs=[
            pl.BlockSpec((N_TEST, FEAT), lambda j, i: (0, 0)),
            pl.BlockSpec((N_TEST, FEAT), lambda j, i: (0, 0)),
            pl.BlockSpec((TILE, FEAT), lambda j, i: (j, 0)),
        ],
        out_specs=[
            pl.BlockSpec((QBLK, CPT, LANE), lambda j, i: (i, j, 0)),
            pl.BlockSpec((KNN, QBLK), lambda j, i: (0, i)),
        ],
        out_shape=[
            jax.ShapeDtypeStruct((N_TEST, N_CHUNKS, LANE), jnp.float32),
            jax.ShapeDtypeStruct((KNN, N_TEST), jnp.int32),
        ],
        scratch_shapes=[
            pltpu.VMEM((N_QBLK, N_TILES, CPT, QBLK), jnp.float32),
            pltpu.VMEM((1, TILE), jnp.float32),
        ],
        compiler_params=pltpu.CompilerParams(
            dimension_semantics=("arbitrary", "arbitrary")),
    )(h_test, h_test2, h_train_pad)


def _sc_gather(t_s, t_p0, t_p1, idx_s, idx_p):
    n_rows = N_TEST * KNN
    mesh = plsc.VectorSubcoreMesh(core_axis_name="c", subcore_axis_name="s")

    nbuf = 4

    @functools.partial(
        pl.kernel, mesh=mesh,
        out_type=[
            jax.ShapeDtypeStruct((n_rows, LANE), jnp.float32),
            jax.ShapeDtypeStruct((n_rows, LANE), jnp.float32),
            jax.ShapeDtypeStruct((n_rows, LANE), jnp.float32),
        ],
        scratch_types=(
            [pltpu.VMEM((N_SUB, LANE), jnp.int32),
             pltpu.VMEM((N_SUB, LANE), jnp.int32)]
            + [pltpu.VMEM((LANE, LANE), jnp.float32)] * nbuf
            + [pltpu.SemaphoreType.DMA] * (2 * nbuf)
        ),
    )
    def k2(ts_hbm, tp0_hbm, tp1_hbm, ixs_hbm, ixp_hbm,
           o_s, o_p0, o_p1, ixs_v, ixp_v, *bufsem):
        bufs = bufsem[:nbuf]
        sin = bufsem[nbuf:2 * nbuf]
        sout = bufsem[2 * nbuf:]
        wid = lax.axis_index("s") * NC + lax.axis_index("c")
        pltpu.sync_copy(ixs_hbm.at[wid], ixs_v)
        pltpu.sync_copy(ixp_hbm.at[wid], ixp_v)
        tasks = []
        for t in range(N_SUB):
            tasks.append((ts_hbm, ixs_v, o_s, t))
            tasks.append((tp0_hbm, ixp_v, o_p0, t))
            tasks.append((tp1_hbm, ixp_v, o_p1, t))
        n_tasks = len(tasks)
        inh = [None] * n_tasks
        outh = [None] * n_tasks

        def start_out(k):
            table, ixv, out, t = tasks[k]
            b = k % nbuf
            inh[k].wait()
            base = wid * ROWS_W + t * LANE
            outh[k] = pltpu.async_copy(bufs[b], out.at[pl.ds(base, LANE)],
                                       sout[b])

        for k in range(n_tasks):
            b = k % nbuf
            if k >= nbuf:
                outh[k - nbuf].wait()
            table, ixv, out, t = tasks[k]
            inh[k] = pltpu.async_copy(table.at[ixv.at[t]], bufs[b], sin[b])
            if k >= 1:
                start_out(k - 1)
        start_out(n_tasks - 1)
        for k in range(n_tasks - nbuf, n_tasks):
            outh[k].wait()

    return k2(t_s, t_p0, t_p1, idx_s, idx_p)


def _k3_body(s_ref, p0_ref, p1_ref, o_ref):
    s = s_ref[...]
    iota = lax.broadcasted_iota(jnp.int32, (QBLK, CAND), 1)
    msk = jnp.zeros((QBLK, CAND), jnp.float32)
    for _ in range(KNN):
        mx = jnp.max(s, axis=1, keepdims=True)
        pos = jnp.min(jnp.where(s == mx, iota, CAND), axis=1, keepdims=True)
        hit = iota == pos
        msk = jnp.where(hit, 1.0, msk)
        s = jnp.where(hit, NEG, s)
    o0 = jnp.sum(p0_ref[...] * msk, axis=1) * (1.0 / KNN)
    o1 = jnp.sum(p1_ref[...] * msk, axis=1) * (1.0 / KNN)
    zero = jnp.zeros((6, QBLK), jnp.float32)
    o_ref[...] = jnp.concatenate([o0[None], o1[None], zero], axis=0)


def _k3(cand_s, cand_p0, cand_p1):
    return pl.pallas_call(
        _k3_body,
        grid=(N_QBLK,),
        in_specs=[
            pl.BlockSpec((QBLK, CAND), lambda i: (i, 0)),
            pl.BlockSpec((QBLK, CAND), lambda i: (i, 0)),
            pl.BlockSpec((QBLK, CAND), lambda i: (i, 0)),
        ],
        out_specs=pl.BlockSpec((8, QBLK), lambda i: (0, i)),
        out_shape=jax.ShapeDtypeStruct((8, N_TEST), jnp.float32),
        compiler_params=pltpu.CompilerParams(
            dimension_semantics=("parallel",)),
    )(cand_s, cand_p0, cand_p1)


def kernel(H_test, H_train, p_hat_train, K):
    del K
    pad_rows = jnp.zeros((N_PAD - N_TRAIN, FEAT), jnp.float32)
    pad_rows = pad_rows.at[:, 0].set(1.0e18)
    h_train_pad = jnp.concatenate([H_train, pad_rows], axis=0)
    scores, ids_t = _k1(H_test, H_test * 2.0, h_train_pad)
    chunk_ids = jnp.transpose(ids_t)

    idx_p = chunk_ids.reshape(NW, N_SUB, LANE)
    idx_s = (jnp.arange(N_TEST, dtype=jnp.int32)[:, None] * N_CHUNKS
             + chunk_ids).reshape(NW, N_SUB, LANE)

    t_s = scores.reshape(N_TEST * N_CHUNKS, LANE)
    pp = jnp.pad(p_hat_train, ((0, 0), (0, N_PAD - N_TRAIN)))
    t_p0 = pp[0].reshape(N_CHUNKS, LANE)
    t_p1 = pp[1].reshape(N_CHUNKS, LANE)

    cand_s, cand_p0, cand_p1 = _sc_gather(t_s, t_p0, t_p1, idx_s, idx_p)
    out = _k3(cand_s.reshape(N_TEST, CAND),
              cand_p0.reshape(N_TEST, CAND),
              cand_p1.reshape(N_TEST, CAND))
    return out[:2, :]


# speedup vs baseline: 1.2417x; 1.1026x over previous
import functools

import jax
import jax.numpy as jnp
from jax import lax
from jax.experimental import pallas as pl
from jax.experimental.pallas import tpu as pltpu
from jax.experimental.pallas import tpu_sc as plsc

N_TEST = 1024
N_TRAIN = 100000
FEAT = 64
KNN = 16
LANE = 128
TILE = 2048
N_PAD = 100352
N_TILES = N_PAD // TILE
CPT = TILE // LANE
N_CHUNKS = N_PAD // LANE
QBLK = 128
N_QBLK = N_TEST // QBLK
QB1 = 256
NQB1 = N_TEST // QB1
CAND = KNN * LANE
NEG = -1.0e30
NC = 2
NS = 16
NW = NC * NS
ROWS_W = (N_TEST * KNN) // NW
N_SUB = ROWS_W // LANE


def _k1_body(a_ref, a2_ref, b_ref, s_ref, ids_ref, m_scr, yn_scr):
    j = pl.program_id(0)
    i = pl.program_id(1)
    row0 = pl.multiple_of(i * QB1, QB1)
    a = a_ref[pl.ds(row0, QB1), :]
    a2 = a2_ref[pl.ds(row0, QB1), :]
    b = b_ref[...]

    @pl.when(i == 0)
    def _():
        yn_scr[...] = jnp.sum(b * b, axis=1)[None, :]

    xn = jnp.sum(a * a, axis=1, keepdims=True)
    tcols = []
    for c in range(CPT):
        bc = b[c * LANE:(c + 1) * LANE, :]
        yn_c = yn_scr[:, c * LANE:(c + 1) * LANE]
        dots2 = lax.dot_general(a2, bc, (((1,), (1,)), ((), ())),
                                preferred_element_type=jnp.float32)
        sc = dots2 - (xn + yn_c)
        s_ref[:, c, :] = sc
        tcols.append(jnp.max(sc, axis=1, keepdims=True))
    tmax = jnp.concatenate(tcols, axis=1)
    m_scr[i, j] = jnp.transpose(tmax)

    @pl.when(j == N_TILES - 1)
    def _():
        m = m_scr[i].reshape(N_CHUNKS, QB1)
        iota = lax.broadcasted_iota(jnp.int32, (N_CHUNKS, QB1), 0)
        rows = []
        for _ in range(KNN):
            mx = jnp.max(m, axis=0, keepdims=True)
            pos = jnp.min(jnp.where(m == mx, iota, N_CHUNKS),
                          axis=0, keepdims=True)
            rows.append(pos)
            m = jnp.where(iota == pos, NEG, m)
        ids_ref[...] = jnp.concatenate(rows, axis=0)


def _k1(h_test, h_test2, h_train_pad):
    return pl.pallas_call(
        _k1_body,
        grid=(N_TILES, NQB1),
        in_specs=[
            pl.BlockSpec((N_TEST, FEAT), lambda j, i: (0, 0)),
            pl.BlockSpec((N_TEST, FEAT), lambda j, i: (0, 0)),
            pl.BlockSpec((TILE, FEAT), lambda j, i: (j, 0)),
        ],
        out_specs=[
            pl.BlockSpec((QB1, CPT, LANE), lambda j, i: (i, j, 0)),
            pl.BlockSpec((KNN, QB1), lambda j, i: (0, i)),
        ],
        out_shape=[
            jax.ShapeDtypeStruct((N_TEST, N_CHUNKS, LANE), jnp.float32),
            jax.ShapeDtypeStruct((KNN, N_TEST), jnp.int32),
        ],
        scratch_shapes=[
            pltpu.VMEM((NQB1, N_TILES, CPT, QB1), jnp.float32),
            pltpu.VMEM((1, TILE), jnp.float32),
        ],
        compiler_params=pltpu.CompilerParams(
            dimension_semantics=("arbitrary", "arbitrary")),
    )(h_test, h_test2, h_train_pad)


def _sc_gather(t_s, t_p0, t_p1, idx_s, idx_p):
    n_rows = N_TEST * KNN
    mesh = plsc.VectorSubcoreMesh(core_axis_name="c", subcore_axis_name="s")

    nbuf = 4

    @functools.partial(
        pl.kernel, mesh=mesh,
        out_type=[
            jax.ShapeDtypeStruct((n_rows, LANE), jnp.float32),
            jax.ShapeDtypeStruct((n_rows, LANE), jnp.float32),
            jax.ShapeDtypeStruct((n_rows, LANE), jnp.float32),
        ],
        scratch_types=(
            [pltpu.VMEM((N_SUB, LANE), jnp.int32),
             pltpu.VMEM((N_SUB, LANE), jnp.int32)]
            + [pltpu.VMEM((LANE, LANE), jnp.float32)] * nbuf
            + [pltpu.SemaphoreType.DMA] * (2 * nbuf)
        ),
    )
    def k2(ts_hbm, tp0_hbm, tp1_hbm, ixs_hbm, ixp_hbm,
           o_s, o_p0, o_p1, ixs_v, ixp_v, *bufsem):
        bufs = bufsem[:nbuf]
        sin = bufsem[nbuf:2 * nbuf]
        sout = bufsem[2 * nbuf:]
        wid = lax.axis_index("s") * NC + lax.axis_index("c")
        pltpu.sync_copy(ixs_hbm.at[wid], ixs_v)
        pltpu.sync_copy(ixp_hbm.at[wid], ixp_v)
        tasks = []
        for t in range(N_SUB):
            tasks.append((ts_hbm, ixs_v, o_s, t))
            tasks.append((tp0_hbm, ixp_v, o_p0, t))
            tasks.append((tp1_hbm, ixp_v, o_p1, t))
        n_tasks = len(tasks)
        inh = [None] * n_tasks
        outh = [None] * n_tasks

        def start_out(k):
            table, ixv, out, t = tasks[k]
            b = k % nbuf
            inh[k].wait()
            base = wid * ROWS_W + t * LANE
            outh[k] = pltpu.async_copy(bufs[b], out.at[pl.ds(base, LANE)],
                                       sout[b])

        for k in range(n_tasks):
            b = k % nbuf
            if k >= nbuf:
                outh[k - nbuf].wait()
            table, ixv, out, t = tasks[k]
            inh[k] = pltpu.async_copy(table.at[ixv.at[t]], bufs[b], sin[b])
            if k >= 1:
                start_out(k - 1)
        start_out(n_tasks - 1)
        for k in range(n_tasks - nbuf, n_tasks):
            outh[k].wait()

    return k2(t_s, t_p0, t_p1, idx_s, idx_p)


def _k3_body(s_ref, p0_ref, p1_ref, o_ref):
    s = s_ref[...]
    iota = lax.broadcasted_iota(jnp.int32, (QBLK, CAND), 1)
    msk = jnp.zeros((QBLK, CAND), jnp.float32)
    for _ in range(KNN):
        mx = jnp.max(s, axis=1, keepdims=True)
        pos = jnp.min(jnp.where(s == mx, iota, CAND), axis=1, keepdims=True)
        hit = iota == pos
        msk = jnp.where(hit, 1.0, msk)
        s = jnp.where(hit, NEG, s)
    o0 = jnp.sum(p0_ref[...] * msk, axis=1) * (1.0 / KNN)
    o1 = jnp.sum(p1_ref[...] * msk, axis=1) * (1.0 / KNN)
    zero = jnp.zeros((6, QBLK), jnp.float32)
    o_ref[...] = jnp.concatenate([o0[None], o1[None], zero], axis=0)


def _k3(cand_s, cand_p0, cand_p1):
    return pl.pallas_call(
        _k3_body,
        grid=(N_QBLK,),
        in_specs=[
            pl.BlockSpec((QBLK, CAND), lambda i: (i, 0)),
            pl.BlockSpec((QBLK, CAND), lambda i: (i, 0)),
            pl.BlockSpec((QBLK, CAND), lambda i: (i, 0)),
        ],
        out_specs=pl.BlockSpec((8, QBLK), lambda i: (0, i)),
        out_shape=jax.ShapeDtypeStruct((8, N_TEST), jnp.float32),
        compiler_params=pltpu.CompilerParams(
            dimension_semantics=("parallel",)),
    )(cand_s, cand_p0, cand_p1)


def kernel(H_test, H_train, p_hat_train, K):
    del K
    pad_rows = jnp.zeros((N_PAD - N_TRAIN, FEAT), jnp.float32)
    pad_rows = pad_rows.at[:, 0].set(1.0e18)
    h_train_pad = jnp.concatenate([H_train, pad_rows], axis=0)
    scores, ids_t = _k1(H_test, H_test * 2.0, h_train_pad)
    chunk_ids = jnp.transpose(ids_t)

    idx_p = chunk_ids.reshape(NW, N_SUB, LANE)
    idx_s = (jnp.arange(N_TEST, dtype=jnp.int32)[:, None] * N_CHUNKS
             + chunk_ids).reshape(NW, N_SUB, LANE)

    t_s = scores.reshape(N_TEST * N_CHUNKS, LANE)
    pp = jnp.pad(p_hat_train, ((0, 0), (0, N_PAD - N_TRAIN)))
    t_p0 = pp[0].reshape(N_CHUNKS, LANE)
    t_p1 = pp[1].reshape(N_CHUNKS, LANE)

    cand_s, cand_p0, cand_p1 = _sc_gather(t_s, t_p0, t_p1, idx_s, idx_p)
    out = _k3(cand_s.reshape(N_TEST, CAND),
              cand_p0.reshape(N_TEST, CAND),
              cand_p1.reshape(N_TEST, CAND))
    return out[:2, :]


# speedup vs baseline: 1.2724x; 1.0247x over previous
import functools

import jax
import jax.numpy as jnp
from jax import lax
from jax.experimental import pallas as pl
from jax.experimental.pallas import tpu as pltpu
from jax.experimental.pallas import tpu_sc as plsc

N_TEST = 1024
N_TRAIN = 100000
FEAT = 64
KNN = 16
LANE = 128
TILE = 2048
N_PAD = 100352
N_TILES = N_PAD // TILE
CPT = TILE // LANE
N_CHUNKS = N_PAD // LANE
QBLK = 128
N_QBLK = N_TEST // QBLK
QB1 = 512
NQB1 = N_TEST // QB1
CAND = KNN * LANE
NEG = -1.0e30
NC = 2
NS = 16
NW = NC * NS
ROWS_W = (N_TEST * KNN) // NW
N_SUB = ROWS_W // LANE


def _k1_body(a_ref, a2_ref, b_ref, s_ref, ids_ref, m_scr, yn_scr):
    j = pl.program_id(0)
    i = pl.program_id(1)
    row0 = pl.multiple_of(i * QB1, QB1)
    a = a_ref[pl.ds(row0, QB1), :]
    a2 = a2_ref[pl.ds(row0, QB1), :]
    b = b_ref[...]

    @pl.when(i == 0)
    def _():
        yn_scr[...] = jnp.sum(b * b, axis=1)[None, :]

    xn = jnp.sum(a * a, axis=1, keepdims=True)
    tcols = []
    for c in range(CPT):
        bc = b[c * LANE:(c + 1) * LANE, :]
        yn_c = yn_scr[:, c * LANE:(c + 1) * LANE]
        dots2 = lax.dot_general(a2, bc, (((1,), (1,)), ((), ())),
                                preferred_element_type=jnp.float32)
        sc = dots2 - (xn + yn_c)
        s_ref[:, c, :] = sc
        tcols.append(jnp.max(sc, axis=1, keepdims=True))
    tmax = jnp.concatenate(tcols, axis=1)
    m_scr[i, j] = jnp.transpose(tmax)

    @pl.when(j == N_TILES - 1)
    def _():
        m = m_scr[i].reshape(N_CHUNKS, QB1)
        iota = lax.broadcasted_iota(jnp.int32, (N_CHUNKS, QB1), 0)
        rows = []
        for _ in range(KNN):
            mx = jnp.max(m, axis=0, keepdims=True)
            pos = jnp.min(jnp.where(m == mx, iota, N_CHUNKS),
                          axis=0, keepdims=True)
            rows.append(pos)
            m = jnp.where(iota == pos, NEG, m)
        ids_ref[...] = jnp.concatenate(rows, axis=0)


def _k1(h_test, h_test2, h_train_pad):
    return pl.pallas_call(
        _k1_body,
        grid=(N_TILES, NQB1),
        in_specs=[
            pl.BlockSpec((N_TEST, FEAT), lambda j, i: (0, 0)),
            pl.BlockSpec((N_TEST, FEAT), lambda j, i: (0, 0)),
            pl.BlockSpec((TILE, FEAT), lambda j, i: (j, 0)),
        ],
        out_specs=[
            pl.BlockSpec((QB1, CPT, LANE), lambda j, i: (i, j, 0)),
            pl.BlockSpec((KNN, QB1), lambda j, i: (0, i)),
        ],
        out_shape=[
            jax.ShapeDtypeStruct((N_TEST, N_CHUNKS, LANE), jnp.float32),
            jax.ShapeDtypeStruct((KNN, N_TEST), jnp.int32),
        ],
        scratch_shapes=[
            pltpu.VMEM((NQB1, N_TILES, CPT, QB1), jnp.float32),
            pltpu.VMEM((1, TILE), jnp.float32),
        ],
        compiler_params=pltpu.CompilerParams(
            dimension_semantics=("arbitrary", "arbitrary")),
    )(h_test, h_test2, h_train_pad)


def _sc_gather(t_s, t_p0, t_p1, idx_s, idx_p):
    n_rows = N_TEST * KNN
    mesh = plsc.VectorSubcoreMesh(core_axis_name="c", subcore_axis_name="s")

    nbuf = 4

    @functools.partial(
        pl.kernel, mesh=mesh,
        out_type=[
            jax.ShapeDtypeStruct((n_rows, LANE), jnp.float32),
            jax.ShapeDtypeStruct((n_rows, LANE), jnp.float32),
            jax.ShapeDtypeStruct((n_rows, LANE), jnp.float32),
        ],
        scratch_types=(
            [pltpu.VMEM((N_SUB, LANE), jnp.int32),
             pltpu.VMEM((N_SUB, LANE), jnp.int32)]
            + [pltpu.VMEM((LANE, LANE), jnp.float32)] * nbuf
            + [pltpu.SemaphoreType.DMA] * (2 * nbuf)
        ),
    )
    def k2(ts_hbm, tp0_hbm, tp1_hbm, ixs_hbm, ixp_hbm,
           o_s, o_p0, o_p1, ixs_v, ixp_v, *bufsem):
        bufs = bufsem[:nbuf]
        sin = bufsem[nbuf:2 * nbuf]
        sout = bufsem[2 * nbuf:]
        wid = lax.axis_index("s") * NC + lax.axis_index("c")
        pltpu.sync_copy(ixs_hbm.at[wid], ixs_v)
        pltpu.sync_copy(ixp_hbm.at[wid], ixp_v)
        tasks = []
        for t in range(N_SUB):
            tasks.append((ts_hbm, ixs_v, o_s, t))
            tasks.append((tp0_hbm, ixp_v, o_p0, t))
            tasks.append((tp1_hbm, ixp_v, o_p1, t))
        n_tasks = len(tasks)
        inh = [None] * n_tasks
        outh = [None] * n_tasks

        def start_out(k):
            table, ixv, out, t = tasks[k]
            b = k % nbuf
            inh[k].wait()
            base = wid * ROWS_W + t * LANE
            outh[k] = pltpu.async_copy(bufs[b], out.at[pl.ds(base, LANE)],
                                       sout[b])

        for k in range(n_tasks):
            b = k % nbuf
            if k >= nbuf:
                outh[k - nbuf].wait()
            table, ixv, out, t = tasks[k]
            inh[k] = pltpu.async_copy(table.at[ixv.at[t]], bufs[b], sin[b])
            if k >= 1:
                start_out(k - 1)
        start_out(n_tasks - 1)
        for k in range(n_tasks - nbuf, n_tasks):
            outh[k].wait()

    return k2(t_s, t_p0, t_p1, idx_s, idx_p)


def _k3_body(s_ref, p0_ref, p1_ref, o_ref):
    s = s_ref[...]
    iota = lax.broadcasted_iota(jnp.int32, (QBLK, CAND), 1)
    msk = jnp.zeros((QBLK, CAND), jnp.float32)
    for _ in range(KNN):
        mx = jnp.max(s, axis=1, keepdims=True)
        pos = jnp.min(jnp.where(s == mx, iota, CAND), axis=1, keepdims=True)
        hit = iota == pos
        msk = jnp.where(hit, 1.0, msk)
        s = jnp.where(hit, NEG, s)
    o0 = jnp.sum(p0_ref[...] * msk, axis=1) * (1.0 / KNN)
    o1 = jnp.sum(p1_ref[...] * msk, axis=1) * (1.0 / KNN)
    zero = jnp.zeros((6, QBLK), jnp.float32)
    o_ref[...] = jnp.concatenate([o0[None], o1[None], zero], axis=0)


def _k3(cand_s, cand_p0, cand_p1):
    return pl.pallas_call(
        _k3_body,
        grid=(N_QBLK,),
        in_specs=[
            pl.BlockSpec((QBLK, CAND), lambda i: (i, 0)),
            pl.BlockSpec((QBLK, CAND), lambda i: (i, 0)),
            pl.BlockSpec((QBLK, CAND), lambda i: (i, 0)),
        ],
        out_specs=pl.BlockSpec((8, QBLK), lambda i: (0, i)),
        out_shape=jax.ShapeDtypeStruct((8, N_TEST), jnp.float32),
        compiler_params=pltpu.CompilerParams(
            dimension_semantics=("parallel",)),
    )(cand_s, cand_p0, cand_p1)


def kernel(H_test, H_train, p_hat_train, K):
    del K
    pad_rows = jnp.zeros((N_PAD - N_TRAIN, FEAT), jnp.float32)
    pad_rows = pad_rows.at[:, 0].set(1.0e18)
    h_train_pad = jnp.concatenate([H_train, pad_rows], axis=0)
    scores, ids_t = _k1(H_test, H_test * 2.0, h_train_pad)
    chunk_ids = jnp.transpose(ids_t)

    idx_p = chunk_ids.reshape(NW, N_SUB, LANE)
    idx_s = (jnp.arange(N_TEST, dtype=jnp.int32)[:, None] * N_CHUNKS
             + chunk_ids).reshape(NW, N_SUB, LANE)

    t_s = scores.reshape(N_TEST * N_CHUNKS, LANE)
    pp = jnp.pad(p_hat_train, ((0, 0), (0, N_PAD - N_TRAIN)))
    t_p0 = pp[0].reshape(N_CHUNKS, LANE)
    t_p1 = pp[1].reshape(N_CHUNKS, LANE)

    cand_s, cand_p0, cand_p1 = _sc_gather(t_s, t_p0, t_p1, idx_s, idx_p)
    out = _k3(cand_s.reshape(N_TEST, CAND),
              cand_p0.reshape(N_TEST, CAND),
              cand_p1.reshape(N_TEST, CAND))
    return out[:2, :]


# speedup vs baseline: 1.3609x; 1.0695x over previous
import functools

import jax
import jax.numpy as jnp
from jax import lax
from jax.experimental import pallas as pl
from jax.experimental.pallas import tpu as pltpu
from jax.experimental.pallas import tpu_sc as plsc

N_TEST = 1024
N_TRAIN = 100000
FEAT = 64
KNN = 16
LANE = 128
TILE = 2048
N_PAD = 100352
N_TILES = N_PAD // TILE
CPT = TILE // LANE
N_CHUNKS = N_PAD // LANE
QBLK = 128
N_QBLK = N_TEST // QBLK
QB1 = 1024
NQB1 = N_TEST // QB1
CAND = KNN * LANE
NEG = -1.0e30
NC = 2
NS = 16
NW = NC * NS
ROWS_W = (N_TEST * KNN) // NW
N_SUB = ROWS_W // LANE


def _k1_body(a_ref, a2_ref, b_ref, s_ref, ids_ref, m_scr, yn_scr):
    j = pl.program_id(0)
    i = pl.program_id(1)
    row0 = pl.multiple_of(i * QB1, QB1)
    a = a_ref[pl.ds(row0, QB1), :]
    a2 = a2_ref[pl.ds(row0, QB1), :]
    b = b_ref[...]

    @pl.when(i == 0)
    def _():
        yn_scr[...] = jnp.sum(b * b, axis=1)[None, :]

    xn = jnp.sum(a * a, axis=1, keepdims=True)
    tcols = []
    for c in range(CPT):
        bc = b[c * LANE:(c + 1) * LANE, :]
        yn_c = yn_scr[:, c * LANE:(c + 1) * LANE]
        dots2 = lax.dot_general(a2, bc, (((1,), (1,)), ((), ())),
                                preferred_element_type=jnp.float32)
        sc = dots2 - (xn + yn_c)
        s_ref[:, c, :] = sc
        tcols.append(jnp.max(sc, axis=1, keepdims=True))
    tmax = jnp.concatenate(tcols, axis=1)
    m_scr[i, j] = jnp.transpose(tmax)

    @pl.when(j == N_TILES - 1)
    def _():
        m = m_scr[i].reshape(N_CHUNKS, QB1)
        iota = lax.broadcasted_iota(jnp.int32, (N_CHUNKS, QB1), 0)
        rows = []
        for _ in range(KNN):
            mx = jnp.max(m, axis=0, keepdims=True)
            pos = jnp.min(jnp.where(m == mx, iota, N_CHUNKS),
                          axis=0, keepdims=True)
            rows.append(pos)
            m = jnp.where(iota == pos, NEG, m)
        ids_ref[...] = jnp.concatenate(rows, axis=0)


def _k1(h_test, h_test2, h_train_pad):
    return pl.pallas_call(
        _k1_body,
        grid=(N_TILES, NQB1),
        in_specs=[
            pl.BlockSpec((N_TEST, FEAT), lambda j, i: (0, 0)),
            pl.BlockSpec((N_TEST, FEAT), lambda j, i: (0, 0)),
            pl.BlockSpec((TILE, FEAT), lambda j, i: (j, 0)),
        ],
        out_specs=[
            pl.BlockSpec((QB1, CPT, LANE), lambda j, i: (i, j, 0)),
            pl.BlockSpec((KNN, QB1), lambda j, i: (0, i)),
        ],
        out_shape=[
            jax.ShapeDtypeStruct((N_TEST, N_CHUNKS, LANE), jnp.float32),
            jax.ShapeDtypeStruct((KNN, N_TEST), jnp.int32),
        ],
        scratch_shapes=[
            pltpu.VMEM((NQB1, N_TILES, CPT, QB1), jnp.float32),
            pltpu.VMEM((1, TILE), jnp.float32),
        ],
        compiler_params=pltpu.CompilerParams(
            dimension_semantics=("arbitrary", "arbitrary")),
    )(h_test, h_test2, h_train_pad)


def _sc_gather(t_s, t_p0, t_p1, idx_s, idx_p):
    n_rows = N_TEST * KNN
    mesh = plsc.VectorSubcoreMesh(core_axis_name="c", subcore_axis_name="s")

    nbuf = 4

    @functools.partial(
        pl.kernel, mesh=mesh,
        out_type=[
            jax.ShapeDtypeStruct((n_rows, LANE), jnp.float32),
            jax.ShapeDtypeStruct((n_rows, LANE), jnp.float32),
            jax.ShapeDtypeStruct((n_rows, LANE), jnp.float32),
        ],
        scratch_types=(
            [pltpu.VMEM((N_SUB, LANE), jnp.int32),
             pltpu.VMEM((N_SUB, LANE), jnp.int32)]
            + [pltpu.VMEM((LANE, LANE), jnp.float32)] * nbuf
            + [pltpu.SemaphoreType.DMA] * (2 * nbuf)
        ),
    )
    def k2(ts_hbm, tp0_hbm, tp1_hbm, ixs_hbm, ixp_hbm,
           o_s, o_p0, o_p1, ixs_v, ixp_v, *bufsem):
        bufs = bufsem[:nbuf]
        sin = bufsem[nbuf:2 * nbuf]
        sout = bufsem[2 * nbuf:]
        wid = lax.axis_index("s") * NC + lax.axis_index("c")
        pltpu.sync_copy(ixs_hbm.at[wid], ixs_v)
        pltpu.sync_copy(ixp_hbm.at[wid], ixp_v)
        tasks = []
        for t in range(N_SUB):
            tasks.append((ts_hbm, ixs_v, o_s, t))
            tasks.append((tp0_hbm, ixp_v, o_p0, t))
            tasks.append((tp1_hbm, ixp_v, o_p1, t))
        n_tasks = len(tasks)
        inh = [None] * n_tasks
        outh = [None] * n_tasks

        def start_out(k):
            table, ixv, out, t = tasks[k]
            b = k % nbuf
            inh[k].wait()
            base = wid * ROWS_W + t * LANE
            outh[k] = pltpu.async_copy(bufs[b], out.at[pl.ds(base, LANE)],
                                       sout[b])

        for k in range(n_tasks):
            b = k % nbuf
            if k >= nbuf:
                outh[k - nbuf].wait()
            table, ixv, out, t = tasks[k]
            inh[k] = pltpu.async_copy(table.at[ixv.at[t]], bufs[b], sin[b])
            if k >= 1:
                start_out(k - 1)
        start_out(n_tasks - 1)
        for k in range(n_tasks - nbuf, n_tasks):
            outh[k].wait()

    return k2(t_s, t_p0, t_p1, idx_s, idx_p)


def _k3_body(s_ref, p0_ref, p1_ref, o_ref):
    s = s_ref[...]
    iota = lax.broadcasted_iota(jnp.int32, (QBLK, CAND), 1)
    msk = jnp.zeros((QBLK, CAND), jnp.float32)
    for _ in range(KNN):
        mx = jnp.max(s, axis=1, keepdims=True)
        pos = jnp.min(jnp.where(s == mx, iota, CAND), axis=1, keepdims=True)
        hit = iota == pos
        msk = jnp.where(hit, 1.0, msk)
        s = jnp.where(hit, NEG, s)
    o0 = jnp.sum(p0_ref[...] * msk, axis=1) * (1.0 / KNN)
    o1 = jnp.sum(p1_ref[...] * msk, axis=1) * (1.0 / KNN)
    zero = jnp.zeros((6, QBLK), jnp.float32)
    o_ref[...] = jnp.concatenate([o0[None], o1[None], zero], axis=0)


def _k3(cand_s, cand_p0, cand_p1):
    return pl.pallas_call(
        _k3_body,
        grid=(N_QBLK,),
        in_specs=[
            pl.BlockSpec((QBLK, CAND), lambda i: (i, 0)),
            pl.BlockSpec((QBLK, CAND), lambda i: (i, 0)),
            pl.BlockSpec((QBLK, CAND), lambda i: (i, 0)),
        ],
        out_specs=pl.BlockSpec((8, QBLK), lambda i: (0, i)),
        out_shape=jax.ShapeDtypeStruct((8, N_TEST), jnp.float32),
        compiler_params=pltpu.CompilerParams(
            dimension_semantics=("parallel",)),
    )(cand_s, cand_p0, cand_p1)


def kernel(H_test, H_train, p_hat_train, K):
    del K
    pad_rows = jnp.zeros((N_PAD - N_TRAIN, FEAT), jnp.float32)
    pad_rows = pad_rows.at[:, 0].set(1.0e18)
    h_train_pad = jnp.concatenate([H_train, pad_rows], axis=0)
    scores, ids_t = _k1(H_test, H_test * 2.0, h_train_pad)
    chunk_ids = jnp.transpose(ids_t)

    idx_p = chunk_ids.reshape(NW, N_SUB, LANE)
    idx_s = (jnp.arange(N_TEST, dtype=jnp.int32)[:, None] * N_CHUNKS
             + chunk_ids).reshape(NW, N_SUB, LANE)

    t_s = scores.reshape(N_TEST * N_CHUNKS, LANE)
    pp = jnp.pad(p_hat_train, ((0, 0), (0, N_PAD - N_TRAIN)))
    t_p0 = pp[0].reshape(N_CHUNKS, LANE)
    t_p1 = pp[1].reshape(N_CHUNKS, LANE)

    cand_s, cand_p0, cand_p1 = _sc_gather(t_s, t_p0, t_p1, idx_s, idx_p)
    out = _k3(cand_s.reshape(N_TEST, CAND),
              cand_p0.reshape(N_TEST, CAND),
              cand_p1.reshape(N_TEST, CAND))
    return out[:2, :]
